# Initial kernel scaffold; baseline (speedup 1.0000x reference)
import jax
import jax.numpy as jnp
from jax.experimental import pallas as pl


def kernel(x, edge_index, edge_weight, att, Wz, bz, Wr, br, Wh, bh, Lz_w, Lz_b, Lr_w, Lr_b, Lh_w, Lh_b, lin_w, lin_b):
    raise NotImplementedError("write your pallas kernel here")


# speedup vs baseline: 314.9436x; 314.9436x over previous
import functools

import jax
import jax.numpy as jnp
from jax import lax
from jax.experimental import pallas as pl
from jax.experimental.pallas import tpu as pltpu
from jax.experimental.pallas import tpu_sc as plsc

N = 10000
E = 320000
P = 12
OUT = 32

NPAD = 10240
ROW = 16
NW = 32
EPT = E // NW
CHUNK = 2000
NSLICE = NPAD // 16


def _sc_deg_body(dst_hbm, w_hbm, zeros_hbm, out_hbm, dst_v, w_v, acc_sh):
    c = lax.axis_index("c")
    s = lax.axis_index("s")
    wid = s * 2 + c

    pltpu.sync_copy(zeros_hbm, acc_sh.at[pl.ds(s * NSLICE, NSLICE)])
    plsc.subcore_barrier()

    def chunk(k, _):
        base = wid * EPT + k * CHUNK
        pltpu.sync_copy(dst_hbm.at[pl.ds(base, CHUNK)], dst_v)
        pltpu.sync_copy(w_hbm.at[pl.ds(base, CHUNK)], w_v)
        pltpu.sync_copy(w_v, acc_sh.at[dst_v], add=True)
        return 0

    lax.fori_loop(0, EPT // CHUNK, chunk, 0)
    plsc.subcore_barrier()
    pltpu.sync_copy(acc_sh.at[pl.ds(s * NSLICE, NSLICE)],
                    out_hbm.at[c, pl.ds(s * NSLICE, NSLICE)])


def _sc_deg(dst, w, zeros_n):
    mesh = plsc.VectorSubcoreMesh(core_axis_name="c", subcore_axis_name="s")
    f = functools.partial(
        pl.kernel, mesh=mesh,
        out_type=jax.ShapeDtypeStruct((2, NPAD), jnp.float32),
        scratch_types=[
            pltpu.VMEM((CHUNK,), jnp.int32),
            pltpu.VMEM((CHUNK,), jnp.float32),
            pltpu.VMEM_SHARED((NPAD,), jnp.float32),
        ],
        compiler_params=pltpu.CompilerParams(needs_layout_passes=False, use_tc_tiling_on_sc=False),
    )(_sc_deg_body)
    return f(dst, w, zeros_n)


def _sc_scatter_body(src_hbm, dst_hbm, w_hbm, y_hbm, zeros_hbm, out_hbm,
                     src_v, dst_v, w_v, rows_v, acc_sh):
    c = lax.axis_index("c")
    s = lax.axis_index("s")
    wid = s * 2 + c

    pltpu.sync_copy(zeros_hbm, acc_sh.at[pl.ds(s * NSLICE, NSLICE)])
    plsc.subcore_barrier()

    iota = lax.iota(jnp.int32, 16)

    def chunk(k, _):
        base = wid * EPT + k * CHUNK
        pltpu.sync_copy(src_hbm.at[pl.ds(base, CHUNK)], src_v)
        pltpu.sync_copy(dst_hbm.at[pl.ds(base, CHUNK)], dst_v)
        pltpu.sync_copy(w_hbm.at[pl.ds(base, CHUNK)], w_v)
        pltpu.sync_copy(y_hbm.at[src_v], rows_v)

        def group(g, _):
            row16 = g * 16 + iota
            w16 = w_v[pl.ds(g * 16, 16)]
            for p in range(P):
                colp = jnp.full((16,), p, jnp.int32)
                v = plsc.load_gather(rows_v, [row16, colp])
                plsc.store_scatter(rows_v, [row16, colp], v * w16)
            return 0

        lax.fori_loop(0, CHUNK // 16, group, 0)

        pltpu.sync_copy(rows_v, acc_sh.at[dst_v], add=True)
        return 0

    lax.fori_loop(0, EPT // CHUNK, chunk, 0)
    plsc.subcore_barrier()
    pltpu.sync_copy(acc_sh.at[pl.ds(s * NSLICE, NSLICE)],
                    out_hbm.at[c, pl.ds(s * NSLICE, NSLICE)])


def _sc_scatter(src, dst, w, y, zeros_rows):
    mesh = plsc.VectorSubcoreMesh(core_axis_name="c", subcore_axis_name="s")
    f = functools.partial(
        pl.kernel, mesh=mesh,
        out_type=jax.ShapeDtypeStruct((2, NPAD, ROW), jnp.float32),
        scratch_types=[
            pltpu.VMEM((CHUNK,), jnp.int32),
            pltpu.VMEM((CHUNK,), jnp.int32),
            pltpu.VMEM((CHUNK,), jnp.float32),
            pltpu.VMEM((CHUNK, ROW), jnp.float32),
            pltpu.VMEM_SHARED((NPAD, ROW), jnp.float32),
        ],
        compiler_params=pltpu.CompilerParams(needs_layout_passes=False, use_tc_tiling_on_sc=False),
    )(_sc_scatter_body)
    return f(src, dst, w, y, zeros_rows)


def _tc_prep_body(deg_ref, x_ref, y_ref, u_ref):
    d = deg_ref[0, :] + deg_ref[1, :] + 1.0
    r = lax.rsqrt(d)
    r = r * (1.5 - 0.5 * d * r * r)
    dinv = jnp.where(d > 0, r, 0.0)
    y_ref[...] = dinv[:, None] * x_ref[...]
    u_ref[...] = jnp.broadcast_to(dinv[:, None], x_ref.shape)


def _tc_prep(deg_parts, x_pad):
    blk = 1024
    grid = (NPAD // blk,)
    return pl.pallas_call(
        _tc_prep_body,
        grid=grid,
        in_specs=[
            pl.BlockSpec((2, blk), lambda i: (0, i)),
            pl.BlockSpec((blk, ROW), lambda i: (i, 0)),
        ],
        out_specs=[
            pl.BlockSpec((blk, ROW), lambda i: (i, 0)),
            pl.BlockSpec((blk, ROW), lambda i: (i, 0)),
        ],
        out_shape=[
            jax.ShapeDtypeStruct((NPAD, ROW), jnp.float32),
            jax.ShapeDtypeStruct((NPAD, ROW), jnp.float32),
        ],
    )(deg_parts, x_pad)


def _tc_final_body(t_ref, y_ref, u_ref, par_ref, out_ref):
    t = t_ref[0] + t_ref[1] + y_ref[...]
    svals = u_ref[...] * t
    uz = par_ref[0:1, :]
    cz0 = par_ref[1:2, :]
    uh = par_ref[2:3, :]
    ch0 = par_ref[3:4, :]
    acc = jnp.zeros((svals.shape[0], OUT), jnp.float32)
    for p in range(P):
        sp = svals[:, p:p + 1]
        z = jax.nn.sigmoid(sp * uz + cz0)
        ht = jnp.tanh(sp * uh + ch0)
        acc = acc + par_ref[5, p] * (1.0 - z) * ht
    h = jnp.maximum(acc, 0.0)
    out_ref[...] = (jnp.sum(h * par_ref[4:5, :], axis=1, keepdims=True)
                    + par_ref[6, 0])


def _tc_final(t_parts, y_pad, u_pad, params):
    blk = 1024
    grid = (NPAD // blk,)
    return pl.pallas_call(
        _tc_final_body,
        grid=grid,
        in_specs=[
            pl.BlockSpec((2, blk, ROW), lambda i: (0, i, 0)),
            pl.BlockSpec((blk, ROW), lambda i: (i, 0)),
            pl.BlockSpec((blk, ROW), lambda i: (i, 0)),
            pl.BlockSpec((8, OUT), lambda i: (0, 0)),
        ],
        out_specs=pl.BlockSpec((blk, 1), lambda i: (i, 0)),
        out_shape=jax.ShapeDtypeStruct((NPAD, 1), jnp.float32),
    )(t_parts, y_pad, u_pad, params)


def kernel(x, edge_index, edge_weight, att, Wz, bz, Wr, br, Wh, bh,
           Lz_w, Lz_b, Lr_w, Lr_b, Lh_w, Lh_b, lin_w, lin_b):
    src = edge_index[0]
    dst = edge_index[1]

    zeros_n = jnp.zeros((NSLICE,), jnp.float32)
    zeros_rows = jnp.zeros((NSLICE, ROW), jnp.float32)

    deg_parts = _sc_deg(dst, edge_weight, zeros_n)

    x_pad = jnp.zeros((NPAD, ROW), jnp.float32).at[:N, :P].set(x)
    y_pad, u_pad = _tc_prep(deg_parts, x_pad)

    t_parts = _sc_scatter(src, dst, edge_weight, y_pad, zeros_rows)

    probs = jax.nn.softmax(att)
    uz = Lz_w[:, :OUT] @ Wz[0]
    cz0 = Lz_w[:, :OUT] @ bz + Lz_b
    uh = Lh_w[:, :OUT] @ Wh[0]
    ch0 = Lh_w[:, :OUT] @ bh + Lh_b
    params = jnp.stack([
        uz, cz0, uh, ch0, lin_w[0],
        jnp.pad(probs, (0, OUT - P)),
        jnp.full((OUT,), lin_b[0], jnp.float32),
        jnp.zeros((OUT,), jnp.float32),
    ])

    out = _tc_final(t_parts, y_pad, u_pad, params)
    return out[:N, :]


# speedup vs baseline: 347.0162x; 1.1018x over previous
import functools

import jax
import jax.numpy as jnp
from jax import lax
from jax.experimental import pallas as pl
from jax.experimental.pallas import tpu as pltpu
from jax.experimental.pallas import tpu_sc as plsc

N = 10000
E = 320000
P = 12
OUT = 32

NPAD = 10240
ROW = 16
NW = 32
EPT = E // NW
CHUNK = 2000
NSLICE = NPAD // 16


def _sc_deg_body(dst_hbm, w_hbm, zeros_hbm, out_hbm, dst_v, w_v, acc_sh):
    c = lax.axis_index("c")
    s = lax.axis_index("s")
    wid = s * 2 + c

    pltpu.sync_copy(zeros_hbm, acc_sh.at[pl.ds(s * NSLICE, NSLICE)])
    plsc.subcore_barrier()

    def chunk(k, _):
        base = wid * EPT + k * CHUNK
        pltpu.sync_copy(dst_hbm.at[pl.ds(base, CHUNK)], dst_v)
        pltpu.sync_copy(w_hbm.at[pl.ds(base, CHUNK)], w_v)
        pltpu.sync_copy(w_v, acc_sh.at[dst_v], add=True)
        return 0

    lax.fori_loop(0, EPT // CHUNK, chunk, 0)
    plsc.subcore_barrier()
    pltpu.sync_copy(acc_sh.at[pl.ds(s * NSLICE, NSLICE)],
                    out_hbm.at[c, pl.ds(s * NSLICE, NSLICE)])


def _sc_deg(dst, w, zeros_n):
    mesh = plsc.VectorSubcoreMesh(core_axis_name="c", subcore_axis_name="s")
    f = functools.partial(
        pl.kernel, mesh=mesh,
        out_type=jax.ShapeDtypeStruct((2, NPAD), jnp.float32),
        scratch_types=[
            pltpu.VMEM((CHUNK,), jnp.int32),
            pltpu.VMEM((CHUNK,), jnp.float32),
            pltpu.VMEM_SHARED((NPAD,), jnp.float32),
        ],
        compiler_params=pltpu.CompilerParams(needs_layout_passes=False, use_tc_tiling_on_sc=False),
    )(_sc_deg_body)
    return f(dst, w, zeros_n)


NCH = EPT // CHUNK


def _sc_scatter_body(src_hbm, dst_hbm, w_hbm, y_hbm, zeros_hbm, out_hbm,
                     src_v, dst_v, w_v, rows_v, acc_sh,
                     sem_in, sem_g, sem_s):
    c = lax.axis_index("c")
    s = lax.axis_index("s")
    wid = s * 2 + c

    pltpu.sync_copy(zeros_hbm, acc_sh.at[pl.ds(s * NSLICE, NSLICE)])

    in_handles = []
    for k in range(NCH):
        base = wid * EPT + k * CHUNK
        hs = pltpu.async_copy(src_hbm.at[pl.ds(base, CHUNK)],
                              src_v.at[k], sem_in)
        hd = pltpu.async_copy(dst_hbm.at[pl.ds(base, CHUNK)],
                              dst_v.at[k], sem_in)
        hw = pltpu.async_copy(w_hbm.at[pl.ds(base, CHUNK)],
                              w_v.at[k], sem_in)
        in_handles.append((hs, hd, hw))
    plsc.subcore_barrier()

    iota = lax.iota(jnp.int32, 16)

    def scale(k, b):
        def group(g, _):
            row16 = g * 16 + iota
            w16 = w_v[k, pl.ds(g * 16, 16)]
            for p in range(P):
                colp = jnp.full((16,), p, jnp.int32)
                v = plsc.load_gather(rows_v.at[b], [row16, colp])
                plsc.store_scatter(rows_v.at[b], [row16, colp], v * w16)
            return 0

        lax.fori_loop(0, CHUNK // 16, group, 0, unroll=2)

    for h in in_handles[0]:
        h.wait()
    g_handles = [pltpu.async_copy(y_hbm.at[src_v.at[0]], rows_v.at[0],
                                  sem_g[0])]
    s_handles = []
    for k in range(NCH):
        b = k % 2
        g_handles[k].wait()
        if k + 1 < NCH:
            for h in in_handles[k + 1]:
                h.wait()
            if k >= 1:
                s_handles[k - 1].wait()
            g_handles.append(
                pltpu.async_copy(y_hbm.at[src_v.at[k + 1]],
                                 rows_v.at[1 - b], sem_g[1 - b]))
        scale(k, b)
        s_handles.append(
            pltpu.async_copy(rows_v.at[b], acc_sh.at[dst_v.at[k]],
                             sem_s[b], add=True))
    s_handles[NCH - 1].wait()
    if NCH >= 2:
        s_handles[NCH - 2].wait()

    plsc.subcore_barrier()
    pltpu.sync_copy(acc_sh.at[pl.ds(s * NSLICE, NSLICE)],
                    out_hbm.at[c, pl.ds(s * NSLICE, NSLICE)])


def _sc_scatter(src, dst, w, y, zeros_rows):
    mesh = plsc.VectorSubcoreMesh(core_axis_name="c", subcore_axis_name="s")
    f = functools.partial(
        pl.kernel, mesh=mesh,
        out_type=jax.ShapeDtypeStruct((2, NPAD, ROW), jnp.float32),
        scratch_types=[
            pltpu.VMEM((NCH, CHUNK), jnp.int32),
            pltpu.VMEM((NCH, CHUNK), jnp.int32),
            pltpu.VMEM((NCH, CHUNK), jnp.float32),
            pltpu.VMEM((2, CHUNK, ROW), jnp.float32),
            pltpu.VMEM_SHARED((NPAD, ROW), jnp.float32),
            pltpu.SemaphoreType.DMA,
            [pltpu.SemaphoreType.DMA, pltpu.SemaphoreType.DMA],
            [pltpu.SemaphoreType.DMA, pltpu.SemaphoreType.DMA],
        ],
        compiler_params=pltpu.CompilerParams(needs_layout_passes=False, use_tc_tiling_on_sc=False),
    )(_sc_scatter_body)
    return f(src, dst, w, y, zeros_rows)


def _tc_prep_body(deg_ref, x_ref, y_ref, u_ref):
    d = deg_ref[0, :] + deg_ref[1, :] + 1.0
    r = lax.rsqrt(d)
    r = r * (1.5 - 0.5 * d * r * r)
    dinv = jnp.where(d > 0, r, 0.0)
    y_ref[...] = dinv[:, None] * x_ref[...]
    u_ref[...] = jnp.broadcast_to(dinv[:, None], x_ref.shape)


def _tc_prep(deg_parts, x_pad):
    blk = 1024
    grid = (NPAD // blk,)
    return pl.pallas_call(
        _tc_prep_body,
        grid=grid,
        in_specs=[
            pl.BlockSpec((2, blk), lambda i: (0, i)),
            pl.BlockSpec((blk, ROW), lambda i: (i, 0)),
        ],
        out_specs=[
            pl.BlockSpec((blk, ROW), lambda i: (i, 0)),
            pl.BlockSpec((blk, ROW), lambda i: (i, 0)),
        ],
        out_shape=[
            jax.ShapeDtypeStruct((NPAD, ROW), jnp.float32),
            jax.ShapeDtypeStruct((NPAD, ROW), jnp.float32),
        ],
    )(deg_parts, x_pad)


def _tc_final_body(t_ref, y_ref, u_ref, par_ref, out_ref):
    t = t_ref[0] + t_ref[1] + y_ref[...]
    svals = u_ref[...] * t
    uz = par_ref[0:1, :]
    cz0 = par_ref[1:2, :]
    uh = par_ref[2:3, :]
    ch0 = par_ref[3:4, :]
    acc = jnp.zeros((svals.shape[0], OUT), jnp.float32)
    for p in range(P):
        sp = svals[:, p:p + 1]
        z = jax.nn.sigmoid(sp * uz + cz0)
        ht = jnp.tanh(sp * uh + ch0)
        acc = acc + par_ref[5, p] * (1.0 - z) * ht
    h = jnp.maximum(acc, 0.0)
    out_ref[...] = (jnp.sum(h * par_ref[4:5, :], axis=1, keepdims=True)
                    + par_ref[6, 0])


def _tc_final(t_parts, y_pad, u_pad, params):
    blk = 1024
    grid = (NPAD // blk,)
    return pl.pallas_call(
        _tc_final_body,
        grid=grid,
        in_specs=[
            pl.BlockSpec((2, blk, ROW), lambda i: (0, i, 0)),
            pl.BlockSpec((blk, ROW), lambda i: (i, 0)),
            pl.BlockSpec((blk, ROW), lambda i: (i, 0)),
            pl.BlockSpec((8, OUT), lambda i: (0, 0)),
        ],
        out_specs=pl.BlockSpec((blk, 1), lambda i: (i, 0)),
        out_shape=jax.ShapeDtypeStruct((NPAD, 1), jnp.float32),
    )(t_parts, y_pad, u_pad, params)


def kernel(x, edge_index, edge_weight, att, Wz, bz, Wr, br, Wh, bh,
           Lz_w, Lz_b, Lr_w, Lr_b, Lh_w, Lh_b, lin_w, lin_b):
    src = edge_index[0]
    dst = edge_index[1]

    zeros_n = jnp.zeros((NSLICE,), jnp.float32)
    zeros_rows = jnp.zeros((NSLICE, ROW), jnp.float32)

    deg_parts = _sc_deg(dst, edge_weight, zeros_n)

    x_pad = jnp.zeros((NPAD, ROW), jnp.float32).at[:N, :P].set(x)
    y_pad, u_pad = _tc_prep(deg_parts, x_pad)

    t_parts = _sc_scatter(src, dst, edge_weight, y_pad, zeros_rows)

    probs = jax.nn.softmax(att)
    uz = Lz_w[:, :OUT] @ Wz[0]
    cz0 = Lz_w[:, :OUT] @ bz + Lz_b
    uh = Lh_w[:, :OUT] @ Wh[0]
    ch0 = Lh_w[:, :OUT] @ bh + Lh_b
    params = jnp.stack([
        uz, cz0, uh, ch0, lin_w[0],
        jnp.pad(probs, (0, OUT - P)),
        jnp.full((OUT,), lin_b[0], jnp.float32),
        jnp.zeros((OUT,), jnp.float32),
    ])

    out = _tc_final(t_parts, y_pad, u_pad, params)
    return out[:N, :]
